# baseline (speedup 1.0000x reference)
import functools

import jax
import jax.numpy as jnp
from jax import lax
from jax.experimental import pallas as pl
from jax.experimental.pallas import tpu as pltpu
from jax.experimental.pallas import tpu_sc as plsc

_EPS = 1e-06
_KS = 4
_NEG = -1000000000.0
_HO = 29
_G = 32
_N = _G * _G
_C = 64
_K = _C * _KS * _KS
_PADROWS = _N + (_KS - 1) * _G + _KS + 4
_ZROW = _N
_OFF = [(d // _KS) * _G + (d % _KS) for d in range(_KS * _KS)]
_QPW = 2 * _G
_SLAB = 168


def _score_kernel(hsh_ref, lsh_ref, gate_row_ref, pmask_ref, gate3_ref,
                  best_ref, cacc_out, a_sc, b_sc, cacc_sc):
    for di in range(_KS):
        r0 = di * _G
        for djp in (0, 2):
            col = (di * _KS + djp) * _C
            a_sc[:, col:col + 2 * _C] = jnp.concatenate(
                [hsh_ref[djp, r0:r0 + _N, :], hsh_ref[djp + 1, r0:r0 + _N, :]],
                axis=1)
            b_sc[:, col:col + 2 * _C] = jnp.concatenate(
                [lsh_ref[djp, r0:r0 + _N, :], lsh_ref[djp + 1, r0:r0 + _N, :]],
                axis=1)
    a = a_sc[...]
    b = b_sc[...]
    st = lax.dot_general(b, a, (((1,), (1,)), ((), ())),
                         precision=lax.Precision.HIGHEST,
                         preferred_element_type=jnp.float32)
    norm = jnp.sqrt(jnp.sum(b * b, axis=1, keepdims=True)) + _EPS
    sn = st / norm
    sn = jnp.where(pmask_ref[...] > 0.5, _NEG, sn)
    m = jnp.max(sn, axis=0, keepdims=True)
    iota_p = lax.broadcasted_iota(jnp.int32, (_N, _N), 0)
    cand = jnp.where(sn == m, iota_p, jnp.int32(2 ** 30))
    best = jnp.min(cand, axis=0, keepdims=True)
    bestg = jnp.where(gate_row_ref[...] > 0.5, best, jnp.int32(_ZROW))
    ident = (lax.broadcasted_iota(jnp.int32, (_N, _N), 0) ==
             lax.broadcasted_iota(jnp.int32, (_N, _N), 1)).astype(jnp.float32)
    bestcol = lax.dot_general(ident, bestg.astype(jnp.float32),
                              (((1,), (1,)), ((), ())),
                              precision=lax.Precision.HIGHEST,
                              preferred_element_type=jnp.float32)
    io_d = lax.broadcasted_iota(jnp.int32, (1, 16), 1)
    offrow = (io_d // _KS) * _G + io_d % _KS
    best_ref[...] = bestcol.astype(jnp.int32) + offrow
    cacc_sc[...] = jnp.zeros((_G, _G, _C), jnp.float32)
    g3 = gate3_ref[0:_HO, 0:_HO, :]
    for d in range(_KS * _KS):
        di, dj = d // _KS, d % _KS
        cacc_sc[di:di + _HO, dj:dj + _HO, :] += g3
    cacc_out[...] = cacc_sc[...].reshape(_N, _C)


def _sc_kernel(lpad_hbm, idx_hbm, slabs_hbm, idx_v, pbuf, slab, sem):
    cid = lax.axis_index("c")
    sid = lax.axis_index("s")

    @pl.when(cid == 0)
    def _():
        base = sid * _QPW
        def zslab(r, c):
            for cc in range(4):
                slab[r, cc * 16:(cc + 1) * 16] = jnp.zeros((16,), jnp.float32)
            return c
        lax.fori_loop(0, 192, zslab, 0)
        @pl.when(sid < 2)
        def _zguard():
            pltpu.sync_copy(slab.at[pl.ds(0, 192)],
                            slabs_hbm.at[pl.ds(sid * 192, 192)])
        for k in range(8):
            pltpu.sync_copy(idx_hbm.at[pl.ds((base + k * 8) * 16, 128)],
                            idx_v)
            pltpu.async_copy(lpad_hbm.at[idx_v], pbuf, sem).wait()

            def accum(jj, c, k=k):
                for d in range(_KS * _KS):
                    row = k * 8 + jj + _OFF[d]
                    src = jj * 16 + d
                    for cc in range(4):
                        sl = pl.ds(cc * 16, 16)
                        slab[row, sl] += pbuf[src, sl]
                return c
            lax.fori_loop(0, 8, accum, 0)
        pltpu.sync_copy(slab.at[pl.ds(0, 192)],
                        slabs_hbm.at[pl.ds((sid + 2) * 192, 192)])


def _blend_kernel(slabs_ref, cacc_ref, lpad_ref, out_ref):
    for w in range(16):
        own = slabs_ref[(w + 2) * 192:(w + 2) * 192 + _QPW, :]
        p1 = slabs_ref[(w + 1) * 192 + _QPW:(w + 1) * 192 + 2 * _QPW, :]
        p2 = slabs_ref[w * 192 + 2 * _QPW:w * 192 + 3 * _QPW, :]
        acc = own + p1 + p2
        cnt = cacc_ref[w * _QPW:(w + 1) * _QPW, :]
        low = lpad_ref[w * _QPW:(w + 1) * _QPW, :]
        out_ref[w * _QPW:(w + 1) * _QPW, :] = jnp.where(
            cnt != 0.0, acc / (cnt + _EPS), low)


def kernel(low_level_features, hight_level_features, mask):
    mask = jnp.asarray(mask)
    lpix = jnp.transpose(low_level_features[0], (1, 2, 0)).reshape(_N, _C)
    hpix = jnp.transpose(hight_level_features[0], (1, 2, 0)).reshape(_N, _C)
    lpad = jnp.pad(lpix, ((0, _PADROWS - _N), (0, 0)))
    hpad = jnp.pad(hpix, ((0, _PADROWS - _N), (0, 0)))
    m00 = mask[:_HO, :_HO]
    m01 = mask[:_HO, _KS:_KS + _HO]
    m10 = mask[_KS:_KS + _HO, :_HO]
    m11 = mask[_KS:_KS + _HO, _KS:_KS + _HO]
    gate = ((m00 != 0) & (m01 != 0) & (m10 != 0) & (m11 != 0)).astype(jnp.float32)
    gate_g = jnp.pad(gate, ((0, _G - _HO), (0, _G - _HO)))
    gate_row = gate_g.reshape(1, _N)
    pm = jnp.pad((mask[:_HO, :_HO] == 1).astype(jnp.float32),
                 ((0, _G - _HO), (0, _G - _HO)), constant_values=1.0)
    pmask = pm.reshape(_N, 1)
    gate3 = jnp.broadcast_to(gate_g[:, :, None], (_G, _G, _C))
    nshift = _N + (_KS - 1) * _G
    hsh = jnp.stack([hpad[dj:dj + nshift, :] for dj in range(_KS)], axis=0)
    lsh = jnp.stack([lpad[dj:dj + nshift, :] for dj in range(_KS)], axis=0)

    idx2d, cacc2d = pl.pallas_call(
        _score_kernel,
        out_shape=(jax.ShapeDtypeStruct((_N, 16), jnp.int32),
                   jax.ShapeDtypeStruct((_N, _C), jnp.float32)),
        scratch_shapes=[pltpu.VMEM((_N, _K), jnp.float32),
                        pltpu.VMEM((_N, _K), jnp.float32),
                        pltpu.VMEM((_G, _G, _C), jnp.float32)],
    )(hsh, lsh, gate_row, pmask, gate3)
    idx_flat = idx2d.reshape(_N * 16)

    sc = functools.partial(
        pl.kernel,
        mesh=plsc.VectorSubcoreMesh(core_axis_name="c", subcore_axis_name="s"),
        out_type=jax.ShapeDtypeStruct((192 * 18, _C), jnp.float32),
        scratch_types=[
            pltpu.VMEM((128,), jnp.int32),
            pltpu.VMEM((128, 128), jnp.float32),
            pltpu.VMEM((192, _C), jnp.float32),
            pltpu.SemaphoreType.DMA,
        ],
    )(_sc_kernel)
    lpad128 = jnp.pad(lpad, ((0, 0), (0, 128 - _C)))
    slabs = sc(lpad128, idx_flat)
    out2d = pl.pallas_call(
        _blend_kernel,
        out_shape=jax.ShapeDtypeStruct((_N, _C), jnp.float32),
    )(slabs, cacc2d, lpad)
    return jnp.transpose(out2d, (1, 0)).reshape(1, _C, _G, _G)


# speedup vs baseline: 3.1941x; 3.1941x over previous
import jax
import jax.numpy as jnp
from jax import lax
from jax.experimental import pallas as pl
from jax.experimental.pallas import tpu as pltpu

_EPS = 1e-06
_KS = 4
_NEG = -1000000000.0
_HO = 29
_G = 32
_N = _G * _G
_C = 64
_K = _C * _KS * _KS
_PADROWS = _N + (_KS - 1) * _G + _KS + 4


def _shift_kernel(lpad_ref, hsh_ref, lsh_ref, gate_row_ref, pmask_ref,
                  gate3_ref, out_ref, a_sc, b_sc, acc_sc, cacc_sc):
    for di in range(_KS):
        r0 = di * _G
        for djp in (0, 2):
            col = (di * _KS + djp) * _C
            a_sc[:, col:col + 2 * _C] = jnp.concatenate(
                [hsh_ref[djp, r0:r0 + _N, :], hsh_ref[djp + 1, r0:r0 + _N, :]],
                axis=1)
            b_sc[:, col:col + 2 * _C] = jnp.concatenate(
                [lsh_ref[djp, r0:r0 + _N, :], lsh_ref[djp + 1, r0:r0 + _N, :]],
                axis=1)
    a = a_sc[...]
    b = b_sc[...]
    st = lax.dot_general(b, a, (((1,), (1,)), ((), ())),
                         precision=lax.Precision.HIGHEST,
                         preferred_element_type=jnp.float32)
    norm = jnp.sqrt(jnp.sum(b * b, axis=1, keepdims=True)) + _EPS
    sn = st / norm
    sn = jnp.where(pmask_ref[...] > 0.5, _NEG, sn)
    m = jnp.max(sn, axis=0, keepdims=True)
    iota_p = lax.broadcasted_iota(jnp.int32, (_N, _N), 0)
    cand = jnp.where(sn == m, iota_p, jnp.int32(2 ** 30))
    best = jnp.min(cand, axis=0, keepdims=True)
    oh = jnp.where(iota_p == best, 1.0, 0.0) * gate_row_ref[...]
    g = lax.dot_general(oh, b, (((0,), (0,)), ((), ())),
                        preferred_element_type=jnp.float32)
    acc_sc[...] = jnp.zeros((_G, _G, _C), jnp.float32)
    cacc_sc[...] = jnp.zeros((_G, _G, _C), jnp.float32)
    g3 = gate3_ref[0:_HO, 0:_HO, :]
    for d in range(_KS * _KS):
        di, dj = d // _KS, d % _KS
        v = g[:, d * _C:(d + 1) * _C].reshape(_G, _G, _C)[0:_HO, 0:_HO, :]
        acc_sc[di:di + _HO, dj:dj + _HO, :] += v
        cacc_sc[di:di + _HO, dj:dj + _HO, :] += g3
    acc = acc_sc[...]
    cacc = cacc_sc[...]
    low3 = lpad_ref[0:_N, :].reshape(_G, _G, _C)
    res = jnp.where(cacc != 0.0, acc / (cacc + _EPS), low3)
    out_ref[...] = res.reshape(_N, _C)


def kernel(low_level_features, hight_level_features, mask):
    mask = jnp.asarray(mask)
    lpix = jnp.transpose(low_level_features[0], (1, 2, 0)).reshape(_N, _C)
    hpix = jnp.transpose(hight_level_features[0], (1, 2, 0)).reshape(_N, _C)
    lpad = jnp.pad(lpix, ((0, _PADROWS - _N), (0, 0)))
    hpad = jnp.pad(hpix, ((0, _PADROWS - _N), (0, 0)))
    m00 = mask[:_HO, :_HO]
    m01 = mask[:_HO, _KS:_KS + _HO]
    m10 = mask[_KS:_KS + _HO, :_HO]
    m11 = mask[_KS:_KS + _HO, _KS:_KS + _HO]
    gate = ((m00 != 0) & (m01 != 0) & (m10 != 0) & (m11 != 0)).astype(jnp.float32)
    gate_g = jnp.pad(gate, ((0, _G - _HO), (0, _G - _HO)))
    gate_row = gate_g.reshape(1, _N)
    pm = jnp.pad((mask[:_HO, :_HO] == 1).astype(jnp.float32),
                 ((0, _G - _HO), (0, _G - _HO)), constant_values=1.0)
    pmask = pm.reshape(_N, 1)
    gate3 = jnp.broadcast_to(gate_g[:, :, None], (_G, _G, _C))
    nshift = _N + (_KS - 1) * _G
    hsh = jnp.stack([hpad[dj:dj + nshift, :] for dj in range(_KS)], axis=0)
    lsh = jnp.stack([lpad[dj:dj + nshift, :] for dj in range(_KS)], axis=0)

    out = pl.pallas_call(
        _shift_kernel,
        out_shape=jax.ShapeDtypeStruct((_N, _C), jnp.float32),
        scratch_shapes=[pltpu.VMEM((_N, _K), jnp.float32),
                        pltpu.VMEM((_N, _K), jnp.float32),
                        pltpu.VMEM((_G, _G, _C), jnp.float32),
                        pltpu.VMEM((_G, _G, _C), jnp.float32)],
    )(lpad, hsh, lsh, gate_row, pmask, gate3)
    return jnp.transpose(out, (1, 0)).reshape(1, _C, _G, _G)
